# baseline (speedup 1.0000x reference)
import dataclasses
import functools

import jax
import jax.numpy as jnp
from jax import lax
from jax.experimental import pallas as pl
from jax.experimental.pallas import tpu as pltpu
from jax.experimental.pallas import tpu_sc as plsc

OBS = 1000
COND = 100
LAT = 128
BATCH = 16384

_NC, _NS = 2, 16
_NW = _NC * _NS
_BPW = BATCH // _NW
_GCH = 128
_NCH = _BPW // _GCH
_L = 16
_NLC = LAT // _L
_CP = 104
_NPAIR = OBS // 2

_SC_PARAMS = pltpu.CompilerParams()
if "needs_layout_passes" in pltpu.CompilerParams.__dataclass_fields__:
    _SC_PARAMS = dataclasses.replace(_SC_PARAMS, needs_layout_passes=False)


def _k1_body(mat3, table, psum, sin0, sin1, eout0, eout1, acc, shared,
             idx104, semi0, semi1, semo0, semo1):
    cid = lax.axis_index("c")
    sid = lax.axis_index("s")
    wid = sid * _NC + cid
    iota16 = lax.broadcasted_iota(jnp.int32, (_L,), 0)
    sins = (sin0, sin1)
    eouts = (eout0, eout1)
    semis = (semi0, semi1)
    semos = (semo0, semo1)
    for k in range(7):
        off = min(k * _L, _CP - _L)
        idx104[pl.ds(off, _L)] = iota16 + off
    npair = jnp.where(wid < 20, 16, 15).astype(jnp.int32)

    def zero_acc(c, carry):
        for j in range(_NLC):
            acc[c, pl.ds(j * _L, _L)] = jnp.zeros((_L,), jnp.float32)
        return carry

    lax.fori_loop(0, _CP, zero_acc, 0)

    def start_in(i, par):
        p = wid + i * _NW
        pltpu.async_copy(mat3.at[2 * p], sins[par].at[pl.ds(0, COND)],
                         semis[par])
        pltpu.async_copy(mat3.at[2 * p + 1],
                         sins[par].at[pl.ds(COND, COND)], semis[par])

    def wait_in(par):
        pltpu.make_async_copy(mat3.at[0], sins[par].at[pl.ds(0, COND)],
                              semis[par]).wait()
        pltpu.make_async_copy(mat3.at[0], sins[par].at[pl.ds(COND, COND)],
                              semis[par]).wait()

    def wait_out(par):
        pltpu.make_async_copy(eouts[par], table.at[pl.ds(0, 2 * COND)],
                              semos[par]).wait()

    @pl.when(0 < npair)
    def _():
        start_in(0, 0)

    @pl.when(1 < npair)
    def _():
        start_in(1, 1)

    def compute(i, par):
        p = wid + i * _NW
        wait_in(par)

        @pl.when(i >= 2)
        def _():
            wait_out(par)

        sin_v = sins[par]
        eout_v = eouts[par]

        def cbody(c, carry):
            for j in range(_NLC):
                s = pl.ds(j * _L, _L)
                e0 = jnp.exp(sin_v[c, s])
                e1 = jnp.exp(sin_v[c + COND, s])
                acc[c, s] += e0 + e1
                eout_v[c, s] = e0
                eout_v[c + COND, s] = e1
            return carry

        lax.fori_loop(0, COND, cbody, 0)

        @pl.when(i + 2 < npair)
        def _():
            start_in(i + 2, par)

        pltpu.async_copy(eout_v, table.at[pl.ds(p * 2 * COND, 2 * COND)],
                         semos[par])

    def outer(t, carry):
        for par in range(2):
            i = 2 * t + par

            @pl.when(i < npair)
            def _():
                compute(i, par)
        return carry

    lax.fori_loop(0, 8, outer, 0)
    wait_out(0)
    wait_out(1)

    @pl.when(sid == 0)
    def _():
        pltpu.sync_copy(acc, shared)

    plsc.subcore_barrier()

    @pl.when(sid != 0)
    def _():
        pltpu.sync_copy(acc, shared.at[idx104], add=True)

    plsc.subcore_barrier()

    @pl.when(sid == 0)
    def _():
        pltpu.sync_copy(shared, psum.at[cid])


_k1 = functools.partial(
    pl.kernel,
    out_type=(
        jax.ShapeDtypeStruct((OBS * COND, LAT), jnp.float32),
        jax.ShapeDtypeStruct((_NC, _CP, LAT), jnp.float32),
    ),
    mesh=plsc.VectorSubcoreMesh(core_axis_name="c", subcore_axis_name="s"),
    compiler_params=_SC_PARAMS,
    scratch_types=[
        pltpu.VMEM((2 * COND, LAT), jnp.float32),
        pltpu.VMEM((2 * COND, LAT), jnp.float32),
        pltpu.VMEM((2 * COND, LAT), jnp.float32),
        pltpu.VMEM((2 * COND, LAT), jnp.float32),
        pltpu.VMEM((_CP, LAT), jnp.float32),
        pltpu.VMEM_SHARED((_CP, LAT), jnp.float32),
        pltpu.VMEM((_CP,), jnp.int32),
        pltpu.SemaphoreType.DMA,
        pltpu.SemaphoreType.DMA,
        pltpu.SemaphoreType.DMA,
        pltpu.SemaphoreType.DMA,
    ],
)(_k1_body)


def _k2_body(table, obs, cond, psum, out, obs2, cond2, idx_v, rin0, rin1,
             rout0, rout1, p_v, rd_v, semm0, semm1, semo0, semo1):
    wid = lax.axis_index("s") * _NC + lax.axis_index("c")
    base = wid * _BPW
    rins = (rin0, rin1)
    routs = (rout0, rout1)
    semms = (semm0, semm1)
    semos = (semo0, semo1)
    for k in range(_NCH):
        pltpu.sync_copy(obs.at[pl.ds(base + k * _GCH, _GCH)], obs2.at[k])
        pltpu.sync_copy(cond.at[pl.ds(base + k * _GCH, _GCH)], cond2.at[k])
    pltpu.sync_copy(psum.at[0], p_v)

    for k in range(_NCH):
        for i in range(_GCH // _L):
            s = pl.ds(i * _L, _L)
            idx_v[k, s] = obs2[k, s] * COND + cond2[k, s]

    def fire_mat(k, par):
        pltpu.async_copy(table.at[idx_v.at[k]], rins[par], semms[par])

    def wait_mat(par):
        pltpu.make_async_copy(table.at[idx_v.at[0]], rins[par],
                              semms[par]).wait()

    def wait_out(par):
        pltpu.make_async_copy(routs[par], out.at[pl.ds(0, _GCH)],
                              semos[par]).wait()

    fire_mat(0, 0)
    fire_mat(1, 1)

    pltpu.sync_copy(psum.at[1], rd_v)

    def rbody(c, carry):
        for j in range(_NLC):
            s = pl.ds(j * _L, _L)
            rd_v[c, s] = 1.0 / (p_v[c, s] + rd_v[c, s])
        return carry

    lax.fori_loop(0, COND, rbody, 0)

    for k in range(_NCH):
        par = k % 2
        wait_mat(par)

        if k >= 2:
            wait_out(par)

        rin_v = rins[par]
        rout_v = routs[par]

        def gbody(g, carry, rin_v=rin_v, rout_v=rout_v, k=k):
            r16 = lax.broadcasted_iota(jnp.int32, (_L,), 0) + g * _L
            c16 = cond2[k, pl.ds(pl.multiple_of(g * _L, _L), _L)]

            def lbody(l, carry2):
                lane16 = jnp.full((_L,), l, jnp.int32)
                rdv = plsc.load_gather(rd_v, [c16, lane16])
                gv = plsc.load_gather(rin_v, [r16, lane16])
                plsc.store_scatter(rout_v, [r16, lane16], gv * rdv)
                return carry2

            lax.fori_loop(0, LAT, lbody, 0)
            return carry

        lax.fori_loop(0, _GCH // _L, gbody, 0)

        pltpu.async_copy(rout_v, out.at[pl.ds(base + k * _GCH, _GCH)],
                         semos[par])
        if k + 2 < _NCH:
            fire_mat(k + 2, par)

    wait_out(0)
    wait_out(1)


_k2 = functools.partial(
    pl.kernel,
    out_type=jax.ShapeDtypeStruct((BATCH, LAT), jnp.float32),
    mesh=plsc.VectorSubcoreMesh(core_axis_name="c", subcore_axis_name="s"),
    compiler_params=_SC_PARAMS,
    scratch_types=[
        pltpu.VMEM((_NCH, _GCH), jnp.int32),
        pltpu.VMEM((_NCH, _GCH), jnp.int32),
        pltpu.VMEM((_NCH, _GCH), jnp.int32),
        pltpu.VMEM((_GCH, LAT), jnp.float32),
        pltpu.VMEM((_GCH, LAT), jnp.float32),
        pltpu.VMEM((_GCH, LAT), jnp.float32),
        pltpu.VMEM((_GCH, LAT), jnp.float32),
        pltpu.VMEM((_CP, LAT), jnp.float32),
        pltpu.VMEM((_CP, LAT), jnp.float32),
        pltpu.SemaphoreType.DMA,
        pltpu.SemaphoreType.DMA,
        pltpu.SemaphoreType.DMA,
        pltpu.SemaphoreType.DMA,
    ],
)(_k2_body)


def kernel(obs, cond_obs, mat):
    table, psum = _k1(mat)
    return _k2(table, obs, cond_obs, psum)


# speedup vs baseline: 1.7214x; 1.7214x over previous
import dataclasses
import functools

import jax
import jax.numpy as jnp
from jax import lax
from jax.experimental import pallas as pl
from jax.experimental.pallas import tpu as pltpu
from jax.experimental.pallas import tpu_sc as plsc

OBS = 1000
COND = 100
LAT = 128
BATCH = 16384

_NC, _NS = 2, 16
_NW = _NC * _NS
_BPW = BATCH // _NW
_GCH = 128
_NCH = _BPW // _GCH
_L = 16
_NLC = LAT // _L
_CP = 104
_NPAIR = OBS // 2

_SC_PARAMS = pltpu.CompilerParams()
if "needs_layout_passes" in pltpu.CompilerParams.__dataclass_fields__:
    _SC_PARAMS = dataclasses.replace(_SC_PARAMS, needs_layout_passes=False)


def _k1_body(mat3, table, psum, sin0, sin1, eout0, eout1, acc, shared,
             idx104, semi0, semi1, semo0, semo1):
    cid = lax.axis_index("c")
    sid = lax.axis_index("s")
    wid = sid * _NC + cid
    iota16 = lax.broadcasted_iota(jnp.int32, (_L,), 0)
    sins = (sin0, sin1)
    eouts = (eout0, eout1)
    semis = (semi0, semi1)
    semos = (semo0, semo1)
    for k in range(7):
        off = min(k * _L, _CP - _L)
        idx104[pl.ds(off, _L)] = iota16 + off
    npair = jnp.where(wid < 20, 16, 15).astype(jnp.int32)

    def zero_acc(c, carry):
        for j in range(_NLC):
            acc[c, pl.ds(j * _L, _L)] = jnp.zeros((_L,), jnp.float32)
        return carry

    lax.fori_loop(0, _CP, zero_acc, 0)

    def start_in(i, par):
        p = wid + i * _NW
        pltpu.async_copy(mat3.at[2 * p], sins[par].at[pl.ds(0, COND)],
                         semis[par])
        pltpu.async_copy(mat3.at[2 * p + 1],
                         sins[par].at[pl.ds(COND, COND)], semis[par])

    def wait_in(par):
        pltpu.make_async_copy(mat3.at[0], sins[par].at[pl.ds(0, COND)],
                              semis[par]).wait()
        pltpu.make_async_copy(mat3.at[0], sins[par].at[pl.ds(COND, COND)],
                              semis[par]).wait()

    def wait_out(par):
        pltpu.make_async_copy(eouts[par], table.at[pl.ds(0, 2 * COND)],
                              semos[par]).wait()

    @pl.when(0 < npair)
    def _():
        start_in(0, 0)

    @pl.when(1 < npair)
    def _():
        start_in(1, 1)

    def compute(i, par):
        p = wid + i * _NW
        wait_in(par)

        @pl.when(i >= 2)
        def _():
            wait_out(par)

        sin_v = sins[par]
        eout_v = eouts[par]

        def cbody(c, carry):
            for j in range(_NLC):
                s = pl.ds(j * _L, _L)
                e0 = jnp.exp(sin_v[c, s])
                e1 = jnp.exp(sin_v[c + COND, s])
                acc[c, s] += e0 + e1
                eout_v[c, s] = e0
                eout_v[c + COND, s] = e1
            return carry

        lax.fori_loop(0, COND, cbody, 0)

        @pl.when(i + 2 < npair)
        def _():
            start_in(i + 2, par)

        pltpu.async_copy(eout_v, table.at[pl.ds(p * 2 * COND, 2 * COND)],
                         semos[par])

    def outer(t, carry):
        for par in range(2):
            i = 2 * t + par

            @pl.when(i < npair)
            def _():
                compute(i, par)
        return carry

    lax.fori_loop(0, 8, outer, 0)
    wait_out(0)
    wait_out(1)

    @pl.when(sid == 0)
    def _():
        pltpu.sync_copy(acc, shared)

    plsc.subcore_barrier()

    @pl.when(sid != 0)
    def _():
        pltpu.sync_copy(acc, shared.at[idx104], add=True)

    plsc.subcore_barrier()

    @pl.when(sid == 0)
    def _():
        pltpu.sync_copy(shared, psum.at[cid])


_k1 = functools.partial(
    pl.kernel,
    out_type=(
        jax.ShapeDtypeStruct((OBS * COND, LAT), jnp.float32),
        jax.ShapeDtypeStruct((_NC, _CP, LAT), jnp.float32),
    ),
    mesh=plsc.VectorSubcoreMesh(core_axis_name="c", subcore_axis_name="s"),
    compiler_params=_SC_PARAMS,
    scratch_types=[
        pltpu.VMEM((2 * COND, LAT), jnp.float32),
        pltpu.VMEM((2 * COND, LAT), jnp.float32),
        pltpu.VMEM((2 * COND, LAT), jnp.float32),
        pltpu.VMEM((2 * COND, LAT), jnp.float32),
        pltpu.VMEM((_CP, LAT), jnp.float32),
        pltpu.VMEM_SHARED((_CP, LAT), jnp.float32),
        pltpu.VMEM((_CP,), jnp.int32),
        pltpu.SemaphoreType.DMA,
        pltpu.SemaphoreType.DMA,
        pltpu.SemaphoreType.DMA,
        pltpu.SemaphoreType.DMA,
    ],
)(_k1_body)


def _k2_body(table, obs, cond, psum, out, obs2, cond2, idx_v, idxb_v,
             rin0, rin1, rout0, rout1, pach, pbch, semm0, semm1, sema,
             semb, semo0, semo1):
    wid = lax.axis_index("s") * _NC + lax.axis_index("c")
    base = wid * _BPW
    psum2 = psum.reshape(_NC * _CP, LAT)
    rins = (rin0, rin1)
    routs = (rout0, rout1)
    semms = (semm0, semm1)
    semos = (semo0, semo1)
    for k in range(_NCH):
        pltpu.sync_copy(obs.at[pl.ds(base + k * _GCH, _GCH)], obs2.at[k])
        pltpu.sync_copy(cond.at[pl.ds(base + k * _GCH, _GCH)], cond2.at[k])

    for k in range(_NCH):
        for i in range(_GCH // _L):
            s = pl.ds(i * _L, _L)
            idx_v[k, s] = obs2[k, s] * COND + cond2[k, s]
            idxb_v[k, s] = cond2[k, s] + _CP

    def fire_mat(k, par):
        pltpu.async_copy(table.at[idx_v.at[k]], rins[par], semms[par])

    def wait_mat(par):
        pltpu.make_async_copy(table.at[idx_v.at[0]], rins[par],
                              semms[par]).wait()

    def fire_p(k):
        pltpu.async_copy(psum2.at[cond2.at[k]], pach, sema)
        pltpu.async_copy(psum2.at[idxb_v.at[k]], pbch, semb)

    def wait_p():
        pltpu.make_async_copy(psum2.at[cond2.at[0]], pach, sema).wait()
        pltpu.make_async_copy(psum2.at[cond2.at[0]], pbch, semb).wait()

    def wait_out(par):
        pltpu.make_async_copy(routs[par], out.at[pl.ds(0, _GCH)],
                              semos[par]).wait()

    fire_mat(0, 0)
    fire_mat(1, 1)
    fire_p(0)

    for k in range(_NCH):
        par = k % 2
        wait_mat(par)
        wait_p()

        if k >= 2:
            wait_out(par)

        rin_v = rins[par]
        rout_v = routs[par]

        def rowb(r, carry, rin_v=rin_v, rout_v=rout_v):
            for j in range(_NLC):
                s = pl.ds(j * _L, _L)
                rout_v[r, s] = rin_v[r, s] * (
                    1.0 / (pach[r, s] + pbch[r, s]))
            return carry

        lax.fori_loop(0, _GCH, rowb, 0)

        pltpu.async_copy(rout_v, out.at[pl.ds(base + k * _GCH, _GCH)],
                         semos[par])
        if k + 2 < _NCH:
            fire_mat(k + 2, par)
        if k + 1 < _NCH:
            fire_p(k + 1)

    wait_out(0)
    wait_out(1)


_k2 = functools.partial(
    pl.kernel,
    out_type=jax.ShapeDtypeStruct((BATCH, LAT), jnp.float32),
    mesh=plsc.VectorSubcoreMesh(core_axis_name="c", subcore_axis_name="s"),
    compiler_params=_SC_PARAMS,
    scratch_types=[
        pltpu.VMEM((_NCH, _GCH), jnp.int32),
        pltpu.VMEM((_NCH, _GCH), jnp.int32),
        pltpu.VMEM((_NCH, _GCH), jnp.int32),
        pltpu.VMEM((_NCH, _GCH), jnp.int32),
        pltpu.VMEM((_GCH, LAT), jnp.float32),
        pltpu.VMEM((_GCH, LAT), jnp.float32),
        pltpu.VMEM((_GCH, LAT), jnp.float32),
        pltpu.VMEM((_GCH, LAT), jnp.float32),
        pltpu.VMEM((_GCH, LAT), jnp.float32),
        pltpu.VMEM((_GCH, LAT), jnp.float32),
        pltpu.SemaphoreType.DMA,
        pltpu.SemaphoreType.DMA,
        pltpu.SemaphoreType.DMA,
        pltpu.SemaphoreType.DMA,
        pltpu.SemaphoreType.DMA,
        pltpu.SemaphoreType.DMA,
    ],
)(_k2_body)


def kernel(obs, cond_obs, mat):
    table, psum = _k1(mat)
    return _k2(table, obs, cond_obs, psum)


# speedup vs baseline: 1.7523x; 1.0180x over previous
import dataclasses
import functools

import jax
import jax.numpy as jnp
from jax import lax
from jax.experimental import pallas as pl
from jax.experimental.pallas import tpu as pltpu
from jax.experimental.pallas import tpu_sc as plsc

OBS = 1000
COND = 100
LAT = 128
BATCH = 16384

_NC, _NS = 2, 16
_NW = _NC * _NS
_BPW = BATCH // _NW
_GCH = 128
_NCH = _BPW // _GCH
_L = 16
_NLC = LAT // _L
_CP = 104
_NPAIR = OBS // 2

_SC_PARAMS = pltpu.CompilerParams()
if "needs_layout_passes" in pltpu.CompilerParams.__dataclass_fields__:
    _SC_PARAMS = dataclasses.replace(_SC_PARAMS, needs_layout_passes=False)


def _k1_body(mat3, table, psum, sin0, sin1, eout0, eout1, acc, shared,
             idx104, semi0, semi1, semo0, semo1):
    cid = lax.axis_index("c")
    sid = lax.axis_index("s")
    wid = sid * _NC + cid
    iota16 = lax.broadcasted_iota(jnp.int32, (_L,), 0)
    sins = (sin0, sin1)
    eouts = (eout0, eout1)
    semis = (semi0, semi1)
    semos = (semo0, semo1)
    for k in range(7):
        off = min(k * _L, _CP - _L)
        idx104[pl.ds(off, _L)] = iota16 + off
    npair = jnp.where(wid < 20, 16, 15).astype(jnp.int32)

    def zero_acc(c, carry):
        for j in range(_NLC):
            acc[c, pl.ds(j * _L, _L)] = jnp.zeros((_L,), jnp.float32)
        return carry

    lax.fori_loop(0, _CP, zero_acc, 0)

    def start_in(i, par):
        p = wid + i * _NW
        pltpu.async_copy(mat3.at[2 * p], sins[par].at[pl.ds(0, COND)],
                         semis[par])
        pltpu.async_copy(mat3.at[2 * p + 1],
                         sins[par].at[pl.ds(COND, COND)], semis[par])

    def wait_in(par):
        pltpu.make_async_copy(mat3.at[0], sins[par].at[pl.ds(0, COND)],
                              semis[par]).wait()
        pltpu.make_async_copy(mat3.at[0], sins[par].at[pl.ds(COND, COND)],
                              semis[par]).wait()

    def wait_out(par):
        pltpu.make_async_copy(eouts[par], table.at[pl.ds(0, 2 * COND)],
                              semos[par]).wait()

    @pl.when(0 < npair)
    def _():
        start_in(0, 0)

    @pl.when(1 < npair)
    def _():
        start_in(1, 1)

    def compute(i, par):
        p = wid + i * _NW
        wait_in(par)

        @pl.when(i >= 2)
        def _():
            wait_out(par)

        sin_v = sins[par]
        eout_v = eouts[par]

        def cbody(c, carry):
            for j in range(_NLC):
                s = pl.ds(j * _L, _L)
                e0 = jnp.exp(sin_v[c, s])
                e1 = jnp.exp(sin_v[c + COND, s])
                acc[c, s] += e0 + e1
                eout_v[c, s] = e0
                eout_v[c + COND, s] = e1
            return carry

        lax.fori_loop(0, COND, cbody, 0)

        @pl.when(i + 2 < npair)
        def _():
            start_in(i + 2, par)

        pltpu.async_copy(eout_v, table.at[pl.ds(p * 2 * COND, 2 * COND)],
                         semos[par])

    def outer(t, carry):
        for par in range(2):
            i = 2 * t + par

            @pl.when(i < npair)
            def _():
                compute(i, par)
        return carry

    lax.fori_loop(0, 8, outer, 0)
    wait_out(0)
    wait_out(1)

    @pl.when(sid == 0)
    def _():
        pltpu.sync_copy(acc, shared)

    plsc.subcore_barrier()

    @pl.when(sid != 0)
    def _():
        pltpu.sync_copy(acc, shared.at[idx104], add=True)

    plsc.subcore_barrier()

    @pl.when(sid == 0)
    def _():
        pltpu.sync_copy(shared, psum.at[cid])


_k1 = functools.partial(
    pl.kernel,
    out_type=(
        jax.ShapeDtypeStruct((OBS * COND, LAT), jnp.float32),
        jax.ShapeDtypeStruct((_NC, _CP, LAT), jnp.float32),
    ),
    mesh=plsc.VectorSubcoreMesh(core_axis_name="c", subcore_axis_name="s"),
    compiler_params=_SC_PARAMS,
    scratch_types=[
        pltpu.VMEM((2 * COND, LAT), jnp.float32),
        pltpu.VMEM((2 * COND, LAT), jnp.float32),
        pltpu.VMEM((2 * COND, LAT), jnp.float32),
        pltpu.VMEM((2 * COND, LAT), jnp.float32),
        pltpu.VMEM((_CP, LAT), jnp.float32),
        pltpu.VMEM_SHARED((_CP, LAT), jnp.float32),
        pltpu.VMEM((_CP,), jnp.int32),
        pltpu.SemaphoreType.DMA,
        pltpu.SemaphoreType.DMA,
        pltpu.SemaphoreType.DMA,
        pltpu.SemaphoreType.DMA,
    ],
)(_k1_body)


def _k2_body(table, obs, cond, psum, out, obs1, cond1, cond2, idx_v,
             idxb_v, rin0, rin1, rout0, rout1, pach, pbch, semm0, semm1,
             sema, semb, semo0, semo1):
    wid = lax.axis_index("s") * _NC + lax.axis_index("c")
    base = wid * _BPW
    psum2 = psum.reshape(_NC * _CP, LAT)
    rins = (rin0, rin1)
    routs = (rout0, rout1)
    semms = (semm0, semm1)
    semos = (semo0, semo1)
    pltpu.sync_copy(obs.at[pl.ds(base, _BPW)], obs1)
    pltpu.sync_copy(cond.at[pl.ds(base, _BPW)], cond1)

    for k in range(_NCH):
        for i in range(_GCH // _L):
            s = pl.ds(i * _L, _L)
            s1 = pl.ds(k * _GCH + i * _L, _L)
            cv = cond1[s1]
            cond2[k, s] = cv
            idx_v[k, s] = obs1[s1] * COND + cv
            idxb_v[k, s] = cv + _CP

    def fire_mat(k, par):
        pltpu.async_copy(table.at[idx_v.at[k]], rins[par], semms[par])

    def wait_mat(par):
        pltpu.make_async_copy(table.at[idx_v.at[0]], rins[par],
                              semms[par]).wait()

    def fire_p(k):
        pltpu.async_copy(psum2.at[cond2.at[k]], pach, sema)
        pltpu.async_copy(psum2.at[idxb_v.at[k]], pbch, semb)

    def wait_p():
        pltpu.make_async_copy(psum2.at[cond2.at[0]], pach, sema).wait()
        pltpu.make_async_copy(psum2.at[cond2.at[0]], pbch, semb).wait()

    def wait_out(par):
        pltpu.make_async_copy(routs[par], out.at[pl.ds(0, _GCH)],
                              semos[par]).wait()

    fire_mat(0, 0)
    fire_mat(1, 1)
    fire_p(0)

    for k in range(_NCH):
        par = k % 2
        wait_mat(par)
        wait_p()

        if k >= 2:
            wait_out(par)

        rin_v = rins[par]
        rout_v = routs[par]

        def rowb(r, carry, rin_v=rin_v, rout_v=rout_v):
            for j in range(_NLC):
                s = pl.ds(j * _L, _L)
                rout_v[r, s] = rin_v[r, s] * (
                    1.0 / (pach[r, s] + pbch[r, s]))
            return carry

        lax.fori_loop(0, _GCH, rowb, 0)

        pltpu.async_copy(rout_v, out.at[pl.ds(base + k * _GCH, _GCH)],
                         semos[par])
        if k + 2 < _NCH:
            fire_mat(k + 2, par)
        if k + 1 < _NCH:
            fire_p(k + 1)

    wait_out(0)
    wait_out(1)


_k2 = functools.partial(
    pl.kernel,
    out_type=jax.ShapeDtypeStruct((BATCH, LAT), jnp.float32),
    mesh=plsc.VectorSubcoreMesh(core_axis_name="c", subcore_axis_name="s"),
    compiler_params=_SC_PARAMS,
    scratch_types=[
        pltpu.VMEM((_BPW,), jnp.int32),
        pltpu.VMEM((_BPW,), jnp.int32),
        pltpu.VMEM((_NCH, _GCH), jnp.int32),
        pltpu.VMEM((_NCH, _GCH), jnp.int32),
        pltpu.VMEM((_NCH, _GCH), jnp.int32),
        pltpu.VMEM((_GCH, LAT), jnp.float32),
        pltpu.VMEM((_GCH, LAT), jnp.float32),
        pltpu.VMEM((_GCH, LAT), jnp.float32),
        pltpu.VMEM((_GCH, LAT), jnp.float32),
        pltpu.VMEM((_GCH, LAT), jnp.float32),
        pltpu.VMEM((_GCH, LAT), jnp.float32),
        pltpu.SemaphoreType.DMA,
        pltpu.SemaphoreType.DMA,
        pltpu.SemaphoreType.DMA,
        pltpu.SemaphoreType.DMA,
        pltpu.SemaphoreType.DMA,
        pltpu.SemaphoreType.DMA,
    ],
)(_k2_body)


def kernel(obs, cond_obs, mat):
    table, psum = _k1(mat)
    return _k2(table, obs, cond_obs, psum)
